# baseline (speedup 1.0000x reference)
import functools

import jax
import jax.numpy as jnp
from jax import lax
from jax.experimental import pallas as pl
from jax.experimental.pallas import tpu as pltpu
from jax.experimental.pallas import tpu_sc as plsc

N = 10000
NPAD = 10240
F = 128
H = 128
G = 64
C = 10
D = 32
NC = 2
NS = 16
NW = NC * NS
EBLK = 128
ICH = 16
RPS = NPAD // NS
CORE0_SHARE_16THS = 8

f32 = jnp.float32


def _sc_segment_sum(xr, src_rows, dst_rows, z128, kpw0, kpw1, with_cnt):
    mesh = plsc.VectorSubcoreMesh(core_axis_name="c", subcore_axis_name="s")
    outs = [jax.ShapeDtypeStruct((NC, NPAD, 128), f32)]
    scratch = [
        pltpu.VMEM((ICH, EBLK), jnp.int32),
        pltpu.VMEM((ICH, EBLK), jnp.int32),
        pltpu.VMEM((EBLK, 128), f32),
        pltpu.VMEM((EBLK, 128), f32),
        pltpu.VMEM_SHARED((NPAD, 128), f32),
        pltpu.SemaphoreType.DMA,
        pltpu.SemaphoreType.DMA,
        pltpu.SemaphoreType.DMA,
        pltpu.SemaphoreType.DMA,
    ]
    if with_cnt:
        outs.append(jax.ShapeDtypeStruct((NC, NPAD, 128), f32))

    @functools.partial(
        pl.kernel,
        out_type=tuple(outs) if with_cnt else outs[0],
        mesh=mesh,
        scratch_types=scratch,
    )
    def k(xr_h, src_h, dst_h, z128_h, *refs):
        if with_cnt:
            (seg_o, cnt_o, src_v, dst_v, rows_a, rows_b, acc_sh,
             sem_a, sem_b, sem_sa, sem_sb) = refs
        else:
            (seg_o, src_v, dst_v, rows_a, rows_b, acc_sh,
             sem_a, sem_b, sem_sa, sem_sb) = refs
        cid = lax.axis_index("c")
        sid = lax.axis_index("s")
        my_kpw = jnp.where(cid == 0, kpw0, kpw1)
        row0 = jnp.where(cid == 0, sid * kpw0, NS * kpw0 + sid * kpw1)
        kpw_max = max(kpw0, kpw1)
        r0 = sid * RPS
        bufs = (rows_a, rows_b)
        sems = (sem_a, sem_b)
        ssems = (sem_sa, sem_sb)
        pltpu.sync_copy(z128_h.at[pl.ds(r0, RPS)], acc_sh.at[pl.ds(r0, RPS)])
        plsc.subcore_barrier()

        @pl.loop(0, kpw_max, step=ICH)
        def _(c0):
            @pl.when(c0 < my_kpw)
            def _():
                pltpu.sync_copy(src_h.at[pl.ds(row0 + c0, ICH)], src_v)
                pltpu.sync_copy(dst_h.at[pl.ds(row0 + c0, ICH)], dst_v)
                h = [None, None]
                hs = [None, None]
                h[0] = pltpu.async_copy(xr_h.at[src_v.at[0]], rows_a, sem_a)
                for j in range(ICH):
                    b = j & 1
                    if j + 1 < ICH:
                        if hs[1 - b] is not None:
                            hs[1 - b].wait()
                        h[1 - b] = pltpu.async_copy(
                            xr_h.at[src_v.at[j + 1]], bufs[1 - b], sems[1 - b])
                    h[b].wait()
                    hs[b] = pltpu.async_copy(bufs[b], acc_sh.at[dst_v.at[j]],
                                             ssems[b], add=True)
                hs[0].wait()
                hs[1].wait()

        plsc.subcore_barrier()
        pltpu.sync_copy(acc_sh.at[pl.ds(r0, RPS)], seg_o.at[cid, pl.ds(r0, RPS)])

        if with_cnt:
            plsc.subcore_barrier()
            pltpu.sync_copy(z128_h.at[pl.ds(r0, RPS)], acc_sh.at[pl.ds(r0, RPS)])

            @pl.loop(0, EBLK)
            def _(i):
                @pl.loop(0, 128 // 16)
                def _(l):
                    rows_a[i, pl.ds(l * 16, 16)] = jnp.full((16,), 1.0, f32)

            plsc.subcore_barrier()

            @pl.loop(0, kpw_max, step=ICH)
            def _(c0):
                @pl.when(c0 < my_kpw)
                def _():
                    pltpu.sync_copy(dst_h.at[pl.ds(row0 + c0, ICH)], dst_v)
                    hs = [pltpu.async_copy(rows_a, acc_sh.at[dst_v.at[j]],
                                           sem_b, add=True)
                          for j in range(ICH)]
                    for hh in hs:
                        hh.wait()

            plsc.subcore_barrier()
            pltpu.sync_copy(acc_sh.at[pl.ds(r0, RPS)], cnt_o.at[cid, pl.ds(r0, RPS)])

    return k(xr, src_rows, dst_rows, z128)


def _tc1_body(x_ref, wrt_ref, wrot_ref, xr_ref, xroot_ref):
    xv = x_ref[...]
    xr_ref[...] = jnp.dot(xv, wrt_ref[...], preferred_element_type=f32)
    xroot_ref[...] = jnp.dot(xv, wrot_ref[...], preferred_element_type=f32)


def _tc2_body(seg_ref, cnt_ref, xroot_ref, b_ref, w2rt_ref, w2rot_ref,
              h1_ref, xr2_ref, hroot2_ref):
    seg = seg_ref[0] + seg_ref[1]
    cnt = cnt_ref[0][:, 0:1] + cnt_ref[1][:, 0:1]
    mean = seg / jnp.maximum(cnt, 1.0)
    h1 = jnp.maximum(mean + b_ref[...] + xroot_ref[...], 0.0)
    h1_ref[...] = h1
    xr2_ref[...] = jnp.dot(h1, w2rt_ref[...], preferred_element_type=f32)
    hroot2_ref[...] = jnp.dot(h1, w2rot_ref[...], preferred_element_type=f32)


def _tc3_body(seg_ref, cnt_ref, hroot2_ref, b2_ref, h1_ref, batch_ref,
              deg_ref, wl1t_ref, bl1_ref, wl2t_ref, bl2_ref, wl4t_ref,
              bl4_ref, wlosst_ref, bloss_ref,
              res1_ref, res3_ref, mse_ref):
    seg = seg_ref[0] + seg_ref[1]
    cnt = cnt_ref[0][:, 0:1] + cnt_ref[1][:, 0:1]
    mean = seg / jnp.maximum(cnt, 1.0)
    h2 = jnp.maximum(mean + b2_ref[...] + hroot2_ref[...], 0.0)
    h1 = h1_ref[...]
    gid = lax.broadcasted_iota(jnp.int32, (G, NPAD), 0)
    P = (gid == batch_ref[...]).astype(f32)
    cnt_g = jnp.maximum(jnp.sum(P, axis=1, keepdims=True), 1.0)
    pool1 = jnp.dot(P, h1, preferred_element_type=f32) / cnt_g
    pool2 = jnp.dot(P, h2, preferred_element_type=f32) / cnt_g
    xcat = jnp.concatenate([pool1, pool2], axis=1)
    x1 = jnp.maximum(jnp.dot(xcat, wl1t_ref[...], preferred_element_type=f32)
                     + bl1_ref[...], 0.0)
    classify = jnp.dot(x1, wl2t_ref[...], preferred_element_type=f32) + bl2_ref[...]
    m = jnp.max(classify, axis=-1, keepdims=True)
    e = classify - m
    res1_ref[...] = e - jnp.log(jnp.sum(jnp.exp(e), axis=-1, keepdims=True))
    res3_ref[...] = (jnp.dot(x1, wl4t_ref[...], preferred_element_type=f32)
                     + bl4_ref[...])
    mse_x = jnp.dot(h2, wlosst_ref[...], preferred_element_type=f32) + bloss_ref[...]
    mask = lax.broadcasted_iota(jnp.int32, (NPAD, 1), 0) < N
    diff = jnp.where(mask, mse_x - deg_ref[...], 0.0)
    mse_ref[...] = jnp.sum(diff * diff, keepdims=True).reshape(1, 1) / N


def kernel(x, edge_index, batch, degree, W1_rel, b1_rel, W1_root, W2_rel,
           b2_rel, W2_root, Wl1, bl1, Wl2, bl2, Wl4, bl4, Wloss, bloss):
    E = edge_index.shape[1]
    rows_ps = -(-E // (NS * EBLK))
    rows_ps = -(-rows_ps // (2 * ICH)) * (2 * ICH)
    kpw0 = (rows_ps * CORE0_SHARE_16THS // 16) // ICH * ICH
    kpw1 = rows_ps - kpw0
    e_pad = NS * rows_ps * EBLK

    x_pad = jnp.pad(x, ((0, NPAD - N), (0, 0)))
    pad_n = e_pad - E
    pad_iota = jnp.arange(pad_n, dtype=jnp.int32)
    src = jnp.concatenate([edge_index[0], pad_iota % N]
                          ).reshape(NS * rows_ps, EBLK)
    dst = jnp.concatenate([edge_index[1], N + pad_iota % (NPAD - N)]
                          ).reshape(NS * rows_ps, EBLK)
    z128 = jnp.zeros((NPAD, 128), f32)
    batch_row = jnp.pad(batch, (0, NPAD - N), constant_values=G).reshape(1, NPAD)
    deg_col = jnp.pad(degree, (0, NPAD - N)).reshape(NPAD, 1)

    nf = jax.ShapeDtypeStruct((NPAD, 128), f32)

    xr1, xroot1 = pl.pallas_call(
        _tc1_body, out_shape=[nf, nf],
    )(x_pad, W1_rel.T, W1_root.T)

    seg1, cnt = _sc_segment_sum(xr1, src, dst, z128, kpw0, kpw1, with_cnt=True)

    h1, xr2, hroot2 = pl.pallas_call(
        _tc2_body, out_shape=[nf, nf, nf],
    )(seg1, cnt, xroot1, b1_rel.reshape(1, H), W2_rel.T, W2_root.T)

    seg2 = _sc_segment_sum(xr2, src, dst, z128, kpw0, kpw1, with_cnt=False)

    res1, res3, mse = pl.pallas_call(
        _tc3_body,
        out_shape=[jax.ShapeDtypeStruct((G, C), f32),
                   jax.ShapeDtypeStruct((G, D), f32),
                   jax.ShapeDtypeStruct((1, 1), f32)],
    )(seg2, cnt, hroot2, b2_rel.reshape(1, H), h1, batch_row, deg_col,
      Wl1.T, bl1.reshape(1, H), Wl2.T, bl2.reshape(1, C), Wl4.T,
      bl4.reshape(1, D), Wloss.T, bloss.reshape(1, 1))

    return (res1, res3, mse.reshape(()))


# speedup vs baseline: 1.2251x; 1.2251x over previous
import functools

import jax
import jax.numpy as jnp
from jax import lax
from jax.experimental import pallas as pl
from jax.experimental.pallas import tpu as pltpu
from jax.experimental.pallas import tpu_sc as plsc

N = 10000
NPAD = 10240
F = 128
H = 128
G = 64
C = 10
D = 32
NC = 2
NS = 16
NW = NC * NS
EBLK = 128
ICH = 16
RPS = NPAD // NS
CORE0_SHARE_16THS = 8

f32 = jnp.float32


def _sc_segment_sum(xr, src_rows, dst_rows, z128, kpw0, kpw1):
    mesh = plsc.VectorSubcoreMesh(core_axis_name="c", subcore_axis_name="s")
    outs = [jax.ShapeDtypeStruct((NC, NPAD, 128), f32)]
    scratch = [
        pltpu.VMEM((ICH, EBLK), jnp.int32),
        pltpu.VMEM((ICH, EBLK), jnp.int32),
        pltpu.VMEM((EBLK, 128), f32),
        pltpu.VMEM((EBLK, 128), f32),
        pltpu.VMEM_SHARED((NPAD, 128), f32),
        pltpu.SemaphoreType.DMA,
        pltpu.SemaphoreType.DMA,
        pltpu.SemaphoreType.DMA,
        pltpu.SemaphoreType.DMA,
    ]
    @functools.partial(
        pl.kernel,
        out_type=outs[0],
        mesh=mesh,
        scratch_types=scratch,
    )
    def k(xr_h, src_h, dst_h, z128_h, *refs):
        (seg_o, src_v, dst_v, rows_a, rows_b, acc_sh,
         sem_a, sem_b, sem_sa, sem_sb) = refs
        cid = lax.axis_index("c")
        sid = lax.axis_index("s")
        my_kpw = jnp.where(cid == 0, kpw0, kpw1)
        row0 = jnp.where(cid == 0, sid * kpw0, NS * kpw0 + sid * kpw1)
        kpw_max = max(kpw0, kpw1)
        r0 = sid * RPS
        bufs = (rows_a, rows_b)
        sems = (sem_a, sem_b)
        ssems = (sem_sa, sem_sb)
        pltpu.sync_copy(z128_h.at[pl.ds(r0, RPS)], acc_sh.at[pl.ds(r0, RPS)])
        plsc.subcore_barrier()

        @pl.loop(0, kpw_max, step=ICH)
        def _(c0):
            @pl.when(c0 < my_kpw)
            def _():
                pltpu.sync_copy(src_h.at[pl.ds(row0 + c0, ICH)], src_v)
                pltpu.sync_copy(dst_h.at[pl.ds(row0 + c0, ICH)], dst_v)
                h = [None, None]
                hs = [None, None]
                h[0] = pltpu.async_copy(xr_h.at[src_v.at[0]], rows_a, sem_a)
                for j in range(ICH):
                    b = j & 1
                    if j + 1 < ICH:
                        if hs[1 - b] is not None:
                            hs[1 - b].wait()
                        h[1 - b] = pltpu.async_copy(
                            xr_h.at[src_v.at[j + 1]], bufs[1 - b], sems[1 - b])
                    h[b].wait()
                    hs[b] = pltpu.async_copy(bufs[b], acc_sh.at[dst_v.at[j]],
                                             ssems[b], add=True)
                hs[0].wait()
                hs[1].wait()

        plsc.subcore_barrier()
        pltpu.sync_copy(acc_sh.at[pl.ds(r0, RPS)], seg_o.at[cid, pl.ds(r0, RPS)])

    return k(xr, src_rows, dst_rows, z128)


CMARK = 8192.0


def _lane127(shape_rows):
    return (lax.broadcasted_iota(jnp.int32, (1, 128), 1) == 127).astype(f32)


def _tc1_body(x_ref, wrt_ref, wrot_ref, xr_ref, xroot_ref):
    xv = x_ref[...]
    xr_ref[...] = (jnp.dot(xv, wrt_ref[...], preferred_element_type=f32)
                   + CMARK * _lane127(None))
    xroot_ref[...] = jnp.dot(xv, wrot_ref[...], preferred_element_type=f32)


def _tc2_body(seg_ref, xroot_ref, b_ref, w2rt_ref, w2rot_ref,
              h1_ref, xr2_ref, hroot2_ref, cnt_ref):
    seg = seg_ref[0] + seg_ref[1]
    cnt = jnp.round(seg[:, 127:128] * (1.0 / CMARK))
    seg = seg - _lane127(None) * (CMARK * cnt)
    mean = seg / jnp.maximum(cnt, 1.0)
    h1 = jnp.maximum(mean + b_ref[...] + xroot_ref[...], 0.0)
    h1_ref[...] = h1
    xr2_ref[...] = jnp.dot(h1, w2rt_ref[...], preferred_element_type=f32)
    hroot2_ref[...] = jnp.dot(h1, w2rot_ref[...], preferred_element_type=f32)
    cnt_ref[...] = cnt


def _tc3_body(seg_ref, cnt_ref, hroot2_ref, b2_ref, h1_ref, batch_ref,
              deg_ref, wl1t_ref, bl1_ref, wl2t_ref, bl2_ref, wl4t_ref,
              bl4_ref, wlosst_ref, bloss_ref,
              res1_ref, res3_ref, mse_ref):
    seg = seg_ref[0] + seg_ref[1]
    mean = seg / jnp.maximum(cnt_ref[...], 1.0)
    h2 = jnp.maximum(mean + b2_ref[...] + hroot2_ref[...], 0.0)
    h1 = h1_ref[...]
    gid = lax.broadcasted_iota(jnp.int32, (G, NPAD), 0)
    P = (gid == batch_ref[...]).astype(f32)
    cnt_g = jnp.maximum(jnp.sum(P, axis=1, keepdims=True), 1.0)
    pool1 = jnp.dot(P, h1, preferred_element_type=f32) / cnt_g
    pool2 = jnp.dot(P, h2, preferred_element_type=f32) / cnt_g
    xcat = jnp.concatenate([pool1, pool2], axis=1)
    x1 = jnp.maximum(jnp.dot(xcat, wl1t_ref[...], preferred_element_type=f32)
                     + bl1_ref[...], 0.0)
    classify = jnp.dot(x1, wl2t_ref[...], preferred_element_type=f32) + bl2_ref[...]
    m = jnp.max(classify, axis=-1, keepdims=True)
    e = classify - m
    res1_ref[...] = e - jnp.log(jnp.sum(jnp.exp(e), axis=-1, keepdims=True))
    res3_ref[...] = (jnp.dot(x1, wl4t_ref[...], preferred_element_type=f32)
                     + bl4_ref[...])
    mse_x = jnp.dot(h2, wlosst_ref[...], preferred_element_type=f32) + bloss_ref[...]
    mask = lax.broadcasted_iota(jnp.int32, (NPAD, 1), 0) < N
    diff = jnp.where(mask, mse_x - deg_ref[...], 0.0)
    mse_ref[...] = jnp.sum(diff * diff, keepdims=True).reshape(1, 1) / N


def kernel(x, edge_index, batch, degree, W1_rel, b1_rel, W1_root, W2_rel,
           b2_rel, W2_root, Wl1, bl1, Wl2, bl2, Wl4, bl4, Wloss, bloss):
    E = edge_index.shape[1]
    rows_ps = -(-E // (NS * EBLK))
    rows_ps = -(-rows_ps // (2 * ICH)) * (2 * ICH)
    kpw0 = (rows_ps * CORE0_SHARE_16THS // 16) // ICH * ICH
    kpw1 = rows_ps - kpw0
    e_pad = NS * rows_ps * EBLK

    x_pad = jnp.pad(x, ((0, NPAD - N), (0, 0)))
    pad_n = e_pad - E
    pad_iota = jnp.arange(pad_n, dtype=jnp.int32)
    src = jnp.concatenate([edge_index[0], pad_iota % N]
                          ).reshape(NS * rows_ps, EBLK)
    dst = jnp.concatenate([edge_index[1], N + pad_iota % (NPAD - N)]
                          ).reshape(NS * rows_ps, EBLK)
    z128 = jnp.zeros((NPAD, 128), f32)
    batch_row = jnp.pad(batch, (0, NPAD - N), constant_values=G).reshape(1, NPAD)
    deg_col = jnp.pad(degree, (0, NPAD - N)).reshape(NPAD, 1)

    nf = jax.ShapeDtypeStruct((NPAD, 128), f32)

    xr1, xroot1 = pl.pallas_call(
        _tc1_body, out_shape=[nf, nf],
    )(x_pad, W1_rel.T, W1_root.T)

    seg1 = _sc_segment_sum(xr1, src, dst, z128, kpw0, kpw1)

    h1, xr2, hroot2, cnt = pl.pallas_call(
        _tc2_body, out_shape=[nf, nf, nf, jax.ShapeDtypeStruct((NPAD, 1), f32)],
    )(seg1, xroot1, b1_rel.reshape(1, H), W2_rel.T, W2_root.T)

    seg2 = _sc_segment_sum(xr2, src, dst, z128, kpw0, kpw1)

    res1, res3, mse = pl.pallas_call(
        _tc3_body,
        out_shape=[jax.ShapeDtypeStruct((G, C), f32),
                   jax.ShapeDtypeStruct((G, D), f32),
                   jax.ShapeDtypeStruct((1, 1), f32)],
    )(seg2, cnt, hroot2, b2_rel.reshape(1, H), h1, batch_row, deg_col,
      Wl1.T, bl1.reshape(1, H), Wl2.T, bl2.reshape(1, C), Wl4.T,
      bl4.reshape(1, D), Wloss.T, bloss.reshape(1, 1))

    return (res1, res3, mse.reshape(()))


# speedup vs baseline: 1.2415x; 1.0134x over previous
import functools

import jax
import jax.numpy as jnp
from jax import lax
from jax.experimental import pallas as pl
from jax.experimental.pallas import tpu as pltpu
from jax.experimental.pallas import tpu_sc as plsc

N = 10000
NPAD = 10240
F = 128
H = 128
G = 64
C = 10
D = 32
NC = 2
NS = 16
NW = NC * NS
EBLK = 128
ICH = 16
RPS = NPAD // NS
CORE0_SHARE_16THS = 8

f32 = jnp.float32


def _sc_segment_sum(xr, sd_rows, z128, kpw0, kpw1):
    mesh = plsc.VectorSubcoreMesh(core_axis_name="c", subcore_axis_name="s")
    outs = [jax.ShapeDtypeStruct((NC, NPAD, 128), f32)]
    scratch = [
        pltpu.VMEM((ICH, 2, EBLK), jnp.int32),
        pltpu.VMEM((EBLK, 128), f32),
        pltpu.VMEM((EBLK, 128), f32),
        pltpu.VMEM_SHARED((NPAD, 128), f32),
        pltpu.SemaphoreType.DMA,
        pltpu.SemaphoreType.DMA,
        pltpu.SemaphoreType.DMA,
        pltpu.SemaphoreType.DMA,
    ]
    @functools.partial(
        pl.kernel,
        out_type=outs[0],
        mesh=mesh,
        scratch_types=scratch,
    )
    def k(xr_h, sd_h, z128_h, *refs):
        (seg_o, sd_v, rows_a, rows_b, acc_sh,
         sem_a, sem_b, sem_sa, sem_sb) = refs
        cid = lax.axis_index("c")
        sid = lax.axis_index("s")
        my_kpw = jnp.where(cid == 0, kpw0, kpw1)
        row0 = jnp.where(cid == 0, sid * kpw0, NS * kpw0 + sid * kpw1)
        kpw_max = max(kpw0, kpw1)
        r0 = sid * RPS
        bufs = (rows_a, rows_b)
        sems = (sem_a, sem_b)
        ssems = (sem_sa, sem_sb)
        pltpu.sync_copy(z128_h.at[pl.ds(r0, RPS)], acc_sh.at[pl.ds(r0, RPS)])
        plsc.subcore_barrier()

        @pl.loop(0, kpw_max, step=ICH)
        def _(c0):
            @pl.when(c0 < my_kpw)
            def _():
                pltpu.sync_copy(sd_h.at[pl.ds(row0 + c0, ICH)], sd_v)
                h = [None, None]
                hs = [None, None]
                h[0] = pltpu.async_copy(xr_h.at[sd_v.at[0, 0]], rows_a, sem_a)
                for j in range(ICH):
                    b = j & 1
                    if j + 1 < ICH:
                        if hs[1 - b] is not None:
                            hs[1 - b].wait()
                        h[1 - b] = pltpu.async_copy(
                            xr_h.at[sd_v.at[j + 1, 0]], bufs[1 - b], sems[1 - b])
                    h[b].wait()
                    hs[b] = pltpu.async_copy(bufs[b], acc_sh.at[sd_v.at[j, 1]],
                                             ssems[b], add=True)
                hs[0].wait()
                hs[1].wait()

        plsc.subcore_barrier()
        pltpu.sync_copy(acc_sh.at[pl.ds(r0, RPS)], seg_o.at[cid, pl.ds(r0, RPS)])

    return k(xr, sd_rows, z128)


CMARK = 8192.0


def _lane127(shape_rows):
    return (lax.broadcasted_iota(jnp.int32, (1, 128), 1) == 127).astype(f32)


def _tc1_body(x_ref, wrt_ref, wrot_ref, xr_ref, xroot_ref):
    xv = x_ref[...]
    xr_ref[...] = (jnp.dot(xv, wrt_ref[...], preferred_element_type=f32)
                   + CMARK * _lane127(None))
    xroot_ref[...] = jnp.dot(xv, wrot_ref[...], preferred_element_type=f32)


def _tc2_body(seg_ref, xroot_ref, b_ref, w2rt_ref, w2rot_ref,
              h1_ref, xr2_ref, hroot2_ref, cnt_ref):
    seg = seg_ref[0] + seg_ref[1]
    cnt = jnp.round(seg[:, 127:128] * (1.0 / CMARK))
    seg = seg - _lane127(None) * (CMARK * cnt)
    mean = seg / jnp.maximum(cnt, 1.0)
    h1 = jnp.maximum(mean + b_ref[...] + xroot_ref[...], 0.0)
    h1_ref[...] = h1
    xr2_ref[...] = jnp.dot(h1, w2rt_ref[...], preferred_element_type=f32)
    hroot2_ref[...] = jnp.dot(h1, w2rot_ref[...], preferred_element_type=f32)
    cnt_ref[...] = cnt


def _tc3_body(seg_ref, cnt_ref, hroot2_ref, b2_ref, h1_ref, batch_ref,
              deg_ref, wl1t_ref, bl1_ref, wl2t_ref, bl2_ref, wl4t_ref,
              bl4_ref, wlosst_ref, bloss_ref,
              res1_ref, res3_ref, mse_ref):
    seg = seg_ref[0] + seg_ref[1]
    mean = seg / jnp.maximum(cnt_ref[...], 1.0)
    h2 = jnp.maximum(mean + b2_ref[...] + hroot2_ref[...], 0.0)
    h1 = h1_ref[...]
    gid = lax.broadcasted_iota(jnp.int32, (G, NPAD), 0)
    P = (gid == batch_ref[...]).astype(f32)
    cnt_g = jnp.maximum(jnp.sum(P, axis=1, keepdims=True), 1.0)
    pool1 = jnp.dot(P, h1, preferred_element_type=f32) / cnt_g
    pool2 = jnp.dot(P, h2, preferred_element_type=f32) / cnt_g
    xcat = jnp.concatenate([pool1, pool2], axis=1)
    x1 = jnp.maximum(jnp.dot(xcat, wl1t_ref[...], preferred_element_type=f32)
                     + bl1_ref[...], 0.0)
    classify = jnp.dot(x1, wl2t_ref[...], preferred_element_type=f32) + bl2_ref[...]
    m = jnp.max(classify, axis=-1, keepdims=True)
    e = classify - m
    res1_ref[...] = e - jnp.log(jnp.sum(jnp.exp(e), axis=-1, keepdims=True))
    res3_ref[...] = (jnp.dot(x1, wl4t_ref[...], preferred_element_type=f32)
                     + bl4_ref[...])
    mse_x = jnp.dot(h2, wlosst_ref[...], preferred_element_type=f32) + bloss_ref[...]
    mask = lax.broadcasted_iota(jnp.int32, (NPAD, 1), 0) < N
    diff = jnp.where(mask, mse_x - deg_ref[...], 0.0)
    mse_ref[...] = jnp.sum(diff * diff, keepdims=True).reshape(1, 1) / N


def kernel(x, edge_index, batch, degree, W1_rel, b1_rel, W1_root, W2_rel,
           b2_rel, W2_root, Wl1, bl1, Wl2, bl2, Wl4, bl4, Wloss, bloss):
    E = edge_index.shape[1]
    rows_ps = -(-E // (NS * EBLK))
    rows_ps = -(-rows_ps // (2 * ICH)) * (2 * ICH)
    kpw0 = (rows_ps * CORE0_SHARE_16THS // 16) // ICH * ICH
    kpw1 = rows_ps - kpw0
    e_pad = NS * rows_ps * EBLK

    x_pad = jnp.pad(x, ((0, NPAD - N), (0, 0)))
    pad_n = e_pad - E
    pad_iota = jnp.arange(pad_n, dtype=jnp.int32)
    src = jnp.concatenate([edge_index[0], pad_iota % N]
                          ).reshape(NS * rows_ps, 1, EBLK)
    dst = jnp.concatenate([edge_index[1], N + pad_iota % (NPAD - N)]
                          ).reshape(NS * rows_ps, 1, EBLK)
    sd = jnp.concatenate([src, dst], axis=1)
    z128 = jnp.zeros((NPAD, 128), f32)
    batch_row = jnp.pad(batch, (0, NPAD - N), constant_values=G).reshape(1, NPAD)
    deg_col = jnp.pad(degree, (0, NPAD - N)).reshape(NPAD, 1)

    nf = jax.ShapeDtypeStruct((NPAD, 128), f32)

    xr1, xroot1 = pl.pallas_call(
        _tc1_body, out_shape=[nf, nf],
    )(x_pad, W1_rel.T, W1_root.T)

    seg1 = _sc_segment_sum(xr1, sd, z128, kpw0, kpw1)

    h1, xr2, hroot2, cnt = pl.pallas_call(
        _tc2_body, out_shape=[nf, nf, nf, jax.ShapeDtypeStruct((NPAD, 1), f32)],
    )(seg1, xroot1, b1_rel.reshape(1, H), W2_rel.T, W2_root.T)

    seg2 = _sc_segment_sum(xr2, sd, z128, kpw0, kpw1)

    res1, res3, mse = pl.pallas_call(
        _tc3_body,
        out_shape=[jax.ShapeDtypeStruct((G, C), f32),
                   jax.ShapeDtypeStruct((G, D), f32),
                   jax.ShapeDtypeStruct((1, 1), f32)],
    )(seg2, cnt, hroot2, b2_rel.reshape(1, H), h1, batch_row, deg_col,
      Wl1.T, bl1.reshape(1, H), Wl2.T, bl2.reshape(1, C), Wl4.T,
      bl4.reshape(1, D), Wloss.T, bloss.reshape(1, 1))

    return (res1, res3, mse.reshape(()))
